# baseline (speedup 1.0000x reference)
import functools

import jax
import jax.numpy as jnp
from jax.experimental import pallas as pl
from jax.experimental.pallas import tpu as pltpu

_MIB = 1024 * 1024


_CONTRACT_LAST = (((1,), (1,)), ((), ()))


def _se_kernel(x_ref, w1_ref, b1_ref, w2_ref, b2_ref, o_ref, *, inv_hw):
    x = x_ref[...]
    pooled = jnp.sum(x.astype(jnp.float32), axis=1) * inv_hw
    h = jax.lax.dot_general(pooled, w1_ref[...], _CONTRACT_LAST,
                            preferred_element_type=jnp.float32)
    h = jnp.maximum(h + b1_ref[...][None, :], 0.0)
    g = jax.lax.dot_general(h, w2_ref[...], _CONTRACT_LAST,
                            preferred_element_type=jnp.float32)
    g = jax.nn.sigmoid(g + b2_ref[...][None, :])
    o_ref[...] = x * g.astype(x.dtype)[:, None, :]


def kernel(x, w1, b1, w2, b2):
    B, C, H, W = x.shape
    Cr = w1.shape[0]
    HW = H * W

    xt = x.transpose(0, 2, 3, 1).reshape(B, HW, C)

    slab = C * HW * jnp.dtype(x.dtype).itemsize
    bblk = 2 if (B % 2 == 0 and B >= 4 and 8 * slab + 8 * _MIB <= 56 * _MIB) else 1
    nb = B // bblk

    out = pl.pallas_call(
        functools.partial(_se_kernel, inv_hw=1.0 / HW),
        out_shape=jax.ShapeDtypeStruct((B, HW, C), x.dtype),
        grid=(nb,),
        in_specs=[
            pl.BlockSpec((bblk, HW, C), lambda i: (i, 0, 0)),
            pl.BlockSpec((Cr, C), lambda i: (0, 0)),
            pl.BlockSpec((Cr,), lambda i: (0,)),
            pl.BlockSpec((C, Cr), lambda i: (0, 0)),
            pl.BlockSpec((C,), lambda i: (0,)),
        ],
        out_specs=pl.BlockSpec((bblk, HW, C), lambda i: (i, 0, 0)),
        compiler_params=pltpu.CompilerParams(
            dimension_semantics=("parallel",),
            vmem_limit_bytes=60 * _MIB),
    )(xt, w1.astype(jnp.float32), b1.astype(jnp.float32),
      w2.astype(jnp.float32), b2.astype(jnp.float32))
    return out.reshape(B, H, W, C).transpose(0, 3, 1, 2)
